# baseline (speedup 1.0000x reference)
import jax
import jax.numpy as jnp
from jax import lax
from jax.experimental import pallas as pl
from jax.experimental.pallas import tpu as pltpu
from jax.experimental.pallas import tpu_sc as plsc
import functools

NC = 2
NS = 16
NW = NC * NS

R = 16384
S = 50
SP = 56
D = 128
R_PER_W = R // NW
RCH = 4
N_PAIR = R_PER_W // (2 * RCH)


def _emb_body(x_hbm, table_hbm, out_hbm, idx_v, rows_a, rows_b,
              gsem_a, gsem_b, ssem_a, ssem_b):
    wid = lax.axis_index("s") * NC + lax.axis_index("c")
    row0 = wid * R_PER_W
    pltpu.sync_copy(x_hbm.at[pl.ds(row0 * SP, R_PER_W * SP)], idx_v)

    def fire_gathers(rows_v, lr0, sem):
        return [pltpu.async_copy(
                    table_hbm.at[idx_v.at[pl.ds((lr0 + r) * SP, S)]],
                    rows_v.at[r], sem)
                for r in range(RCH)]

    @pl.loop(0, N_PAIR)
    def _pair(t):
        la = t * (2 * RCH)
        lb = la + RCH

        @pl.when(t > 0)
        def _():
            pltpu.make_async_copy(rows_a, out_hbm.at[pl.ds(row0 + la, RCH)],
                                  ssem_a).wait()
        ga = fire_gathers(rows_a, la, gsem_a)

        @pl.when(t > 0)
        def _():
            pltpu.make_async_copy(rows_b, out_hbm.at[pl.ds(row0 + lb, RCH)],
                                  ssem_b).wait()
        gb = fire_gathers(rows_b, lb, gsem_b)

        for g in ga:
            g.wait()
        pltpu.async_copy(rows_a, out_hbm.at[pl.ds(row0 + la, RCH)], ssem_a)
        for g in gb:
            g.wait()
        pltpu.async_copy(rows_b, out_hbm.at[pl.ds(row0 + lb, RCH)], ssem_b)

    pltpu.make_async_copy(rows_a, out_hbm.at[pl.ds(row0, RCH)], ssem_a).wait()
    pltpu.make_async_copy(rows_b, out_hbm.at[pl.ds(row0, RCH)], ssem_b).wait()


@functools.partial(jax.jit, static_argnames=())
def _emb_lookup(x, table):
    mesh = plsc.VectorSubcoreMesh(core_axis_name="c", subcore_axis_name="s")
    f = pl.kernel(
        _emb_body,
        out_type=jax.ShapeDtypeStruct((R, S, D), jnp.float32),
        mesh=mesh,
        scratch_types=[
            pltpu.VMEM((R_PER_W * SP,), jnp.int32),
            pltpu.VMEM((RCH, S, D), jnp.float32),
            pltpu.VMEM((RCH, S, D), jnp.float32),
            pltpu.SemaphoreType.DMA,
            pltpu.SemaphoreType.DMA,
            pltpu.SemaphoreType.DMA,
            pltpu.SemaphoreType.DMA,
        ],
    )
    return f(x, table)


def kernel(x, table):
    xp = jnp.pad(x.astype(jnp.int32), ((0, 0), (0, SP - S)))
    return _emb_lookup(xp.reshape(-1), table)


# speedup vs baseline: 1.0200x; 1.0200x over previous
import jax
import jax.numpy as jnp
from jax import lax
from jax.experimental import pallas as pl
from jax.experimental.pallas import tpu as pltpu
from jax.experimental.pallas import tpu_sc as plsc
import functools

NC = 2
NS = 16
NW = NC * NS

R = 16384
S = 50
D = 128
R_PER_W = R // NW
B_PER_W = R_PER_W * S
NBUF = 4
RCH = 4
CH = RCH * S
N_GRP = R_PER_W // (NBUF * RCH)
G_OFF = (0, 96)
G_LEN = (96, 104)


def _emb_body(idx_hbm, table_hbm, out_hbm, idx_v,
              rows_0, rows_1, rows_2, rows_3,
              gsem_0, gsem_1, gsem_2, gsem_3,
              ssem_0, ssem_1, ssem_2, ssem_3):
    rows = (rows_0, rows_1, rows_2, rows_3)
    gsem = (gsem_0, gsem_1, gsem_2, gsem_3)
    ssem = (ssem_0, ssem_1, ssem_2, ssem_3)
    wid = lax.axis_index("s") * NC + lax.axis_index("c")
    row0 = wid * R_PER_W
    pltpu.sync_copy(idx_hbm.at[pl.ds(wid * B_PER_W, B_PER_W)], idx_v)

    def drain_scatters(b, r0):
        for r in range(RCH):
            pltpu.make_async_copy(rows[b].at[pl.ds(r * S, S)],
                                  out_hbm.at[r0 + r], ssem[b]).wait()

    def fire_gathers(b, off):
        return [pltpu.async_copy(table_hbm.at[idx_v.at[pl.ds(off + o, n)]],
                                 rows[b].at[pl.ds(o, n)], gsem[b])
                for o, n in zip(G_OFF, G_LEN)]

    def fire_scatters(b, r0):
        for r in range(RCH):
            pltpu.async_copy(rows[b].at[pl.ds(r * S, S)], out_hbm.at[r0 + r],
                             ssem[b])

    @pl.loop(0, N_GRP)
    def _grp(t):
        base = row0 + t * (NBUF * RCH)
        gs = []
        for b in range(NBUF):
            @pl.when(t > 0)
            def _(b=b):
                drain_scatters(b, base + b * RCH)
            gs.append(fire_gathers(b, (t * NBUF + b) * CH))
        for b in range(NBUF):
            for g in gs[b]:
                g.wait()
            fire_scatters(b, base + b * RCH)

    for b in range(NBUF):
        drain_scatters(b, row0 + b * RCH)


@functools.partial(jax.jit, static_argnames=())
def _emb_lookup(idx_flat, table):
    mesh = plsc.VectorSubcoreMesh(core_axis_name="c", subcore_axis_name="s")
    f = pl.kernel(
        _emb_body,
        out_type=jax.ShapeDtypeStruct((R, S, D), jnp.float32),
        mesh=mesh,
        scratch_types=(
            [pltpu.VMEM((B_PER_W,), jnp.int32)]
            + [pltpu.VMEM((CH, D), jnp.float32) for _ in range(NBUF)]
            + [pltpu.SemaphoreType.DMA for _ in range(2 * NBUF)]
        ),
    )
    return f(idx_flat, table)


def kernel(x, table):
    idx_flat = x.reshape(-1).astype(jnp.int32)
    return _emb_lookup(idx_flat, table)
